# Initial kernel scaffold; baseline (speedup 1.0000x reference)
import jax
import jax.numpy as jnp
from jax.experimental import pallas as pl


def kernel(x, edge_index, W1, b1, W2, b2):
    raise NotImplementedError("write your pallas kernel here")


# speedup vs baseline: 21.4611x; 21.4611x over previous
import dataclasses
import functools

import jax
import jax.numpy as jnp
from jax import lax
from jax.experimental import pallas as pl
from jax.experimental.pallas import tpu as pltpu
from jax.experimental.pallas import tpu_sc as plsc

N = 10000
E = 320000
D = 128
H = 128
C = 40
CP = 64

NC = 2
NS = 16
NW = NC * NS
LANES = 16

NPAD = 10240

K = 80
EPT = E // NW
NCHUNK = EPT // K
RPT = NPAD // NS

EPT_DEG = E // NS
SL = NPAD // NS

_mesh = plsc.VectorSubcoreMesh(core_axis_name="c", subcore_axis_name="s")

_sc_params = pltpu.CompilerParams()
for _f, _v in (("needs_layout_passes", False), ("use_tc_tiling_on_sc", False)):
    if _f in pltpu.CompilerParams.__dataclass_fields__:
        _sc_params = dataclasses.replace(_sc_params, **{_f: _v})


def _deg_dis_kernel(dst_hbm, dis_hbm, dst_v, deg_v, acc_v, tmp_v, deg_sh):
    cid = lax.axis_index("c")
    sid = lax.axis_index("s")

    @pl.when(cid == 0)
    def _():
        @pl.loop(0, NPAD, step=LANES)
        def _(i):
            deg_v[pl.ds(i, LANES)] = jnp.zeros((LANES,), jnp.float32)

        pltpu.sync_copy(dst_hbm.at[pl.ds(sid * EPT_DEG, EPT_DEG)], dst_v)

        @pl.loop(0, EPT_DEG, step=LANES)
        def _(i):
            idx = dst_v[pl.ds(i, LANES)]
            plsc.addupdate_scatter(deg_v, [idx], jnp.ones((LANES,), jnp.float32))

        pltpu.sync_copy(deg_v, deg_sh.at[sid])

    plsc.subcore_barrier()

    @pl.when(cid == 0)
    def _():
        @pl.loop(0, SL, step=LANES)
        def _(i):
            acc_v[pl.ds(i, LANES)] = jnp.zeros((LANES,), jnp.float32)

        @pl.loop(0, NS)
        def _(k):
            pltpu.sync_copy(deg_sh.at[k, pl.ds(sid * SL, SL)], tmp_v)

            @pl.loop(0, SL, step=LANES)
            def _(i):
                acc_v[pl.ds(i, LANES)] = acc_v[pl.ds(i, LANES)] + tmp_v[pl.ds(i, LANES)]

        @pl.loop(0, SL, step=LANES)
        def _(i):
            d = acc_v[pl.ds(i, LANES)] + 1.0
            xh = d * 0.5
            ii = plsc.bitcast(d, jnp.int32)
            ii = jnp.int32(0x5F3759DF) - lax.shift_right_logical(ii, jnp.int32(1))
            yv = plsc.bitcast(ii, jnp.float32)
            yv = yv * (1.5 - xh * yv * yv)
            yv = yv * (1.5 - xh * yv * yv)
            yv = yv * (1.5 - xh * yv * yv)
            acc_v[pl.ds(i, LANES)] = yv

        pltpu.sync_copy(acc_v, dis_hbm.at[pl.ds(sid * SL, SL)])


def _make_deg_dis():
    return pl.kernel(
        _deg_dis_kernel,
        out_type=jax.ShapeDtypeStruct((NPAD,), jnp.float32),
        mesh=_mesh,
        scratch_types=[
            pltpu.VMEM((EPT_DEG,), jnp.int32),
            pltpu.VMEM((NPAD,), jnp.float32),
            pltpu.VMEM((SL,), jnp.float32),
            pltpu.VMEM((SL,), jnp.float32),
            pltpu.VMEM_SHARED((NS, NPAD), jnp.float32),
        ],
        compiler_params=_sc_params,
    )


def _agg_kernel(width, y_hbm, src_hbm, dst_hbm, zeros_hbm, out_hbm,
                src_v, dst_v, rows_v, acc_sh):
    cid = lax.axis_index("c")
    sid = lax.axis_index("s")
    wid = sid * NC + cid

    pltpu.sync_copy(src_hbm.at[wid], src_v)
    pltpu.sync_copy(dst_hbm.at[wid], dst_v)
    pltpu.sync_copy(zeros_hbm, acc_sh.at[pl.ds(sid * RPT, RPT)])
    plsc.subcore_barrier()

    @pl.loop(0, NCHUNK)
    def _(j):
        pltpu.sync_copy(y_hbm.at[src_v.at[j]], rows_v)
        pltpu.sync_copy(rows_v, acc_sh.at[dst_v.at[j]], add=True)

    plsc.subcore_barrier()
    pltpu.sync_copy(acc_sh.at[pl.ds(sid * RPT, RPT)],
                    out_hbm.at[cid, pl.ds(sid * RPT, RPT)])


def _make_agg(width):
    return pl.kernel(
        functools.partial(_agg_kernel, width),
        out_type=jax.ShapeDtypeStruct((NC, NPAD, width), jnp.float32),
        mesh=_mesh,
        scratch_types=[
            pltpu.VMEM((NCHUNK, K), jnp.int32),
            pltpu.VMEM((NCHUNK, K), jnp.int32),
            pltpu.VMEM((K, width), jnp.float32),
            pltpu.VMEM_SHARED((NPAD, width), jnp.float32),
        ],
        compiler_params=_sc_params,
    )


RB = 2048
GRID = NPAD // RB


def _tc1_body(dis_ref, x_ref, w_ref, y_ref):
    xw = jnp.dot(x_ref[...], w_ref[...], preferred_element_type=jnp.float32)
    y_ref[...] = xw * dis_ref[...]


def _tc2_body(dis_ref, p_ref, y_ref, b_ref, w_ref, z_ref):
    pre = dis_ref[...] * (p_ref[0] + p_ref[1] + y_ref[...]) + b_ref[...]
    h = jnp.maximum(pre, 0.0)
    z_ref[...] = jnp.dot(h, w_ref[...], preferred_element_type=jnp.float32) * dis_ref[...]


def _tc3_body(dis_ref, q_ref, z_ref, b_ref, o_ref):
    logits = dis_ref[...] * (q_ref[0] + q_ref[1] + z_ref[...]) + b_ref[...]
    m = jnp.max(logits, axis=1, keepdims=True)
    e = logits - m
    o_ref[...] = e - jnp.log(jnp.sum(jnp.exp(e), axis=1, keepdims=True))


def _tc1(dis2d, x, W1):
    return pl.pallas_call(
        _tc1_body,
        grid=(GRID,),
        in_specs=[
            pl.BlockSpec((RB, 1), lambda i: (i, 0)),
            pl.BlockSpec((RB, D), lambda i: (i, 0)),
            pl.BlockSpec((D, H), lambda i: (0, 0)),
        ],
        out_specs=pl.BlockSpec((RB, H), lambda i: (i, 0)),
        out_shape=jax.ShapeDtypeStruct((NPAD, H), jnp.float32),
    )(dis2d, x, W1)


def _tc2(dis2d, p, y, b1r, W2p):
    return pl.pallas_call(
        _tc2_body,
        grid=(GRID,),
        in_specs=[
            pl.BlockSpec((RB, 1), lambda i: (i, 0)),
            pl.BlockSpec((NC, RB, H), lambda i: (0, i, 0)),
            pl.BlockSpec((RB, H), lambda i: (i, 0)),
            pl.BlockSpec((1, H), lambda i: (0, 0)),
            pl.BlockSpec((H, CP), lambda i: (0, 0)),
        ],
        out_specs=pl.BlockSpec((RB, CP), lambda i: (i, 0)),
        out_shape=jax.ShapeDtypeStruct((NPAD, CP), jnp.float32),
    )(dis2d, p, y, b1r, W2p)


def _tc3(dis2d, q, z, b2r):
    return pl.pallas_call(
        _tc3_body,
        grid=(GRID,),
        in_specs=[
            pl.BlockSpec((RB, 1), lambda i: (i, 0)),
            pl.BlockSpec((NC, RB, CP), lambda i: (0, i, 0)),
            pl.BlockSpec((RB, CP), lambda i: (i, 0)),
            pl.BlockSpec((1, CP), lambda i: (0, 0)),
        ],
        out_specs=pl.BlockSpec((RB, CP), lambda i: (i, 0)),
        out_shape=jax.ShapeDtypeStruct((NPAD, CP), jnp.float32),
    )(dis2d, q, z, b2r)


def kernel(x, edge_index, W1, b1, W2, b2):
    ei = edge_index.astype(jnp.int32)
    src3 = ei[0].reshape(NW, NCHUNK, K)
    dst3 = ei[1].reshape(NW, NCHUNK, K)

    dis_pad = _make_deg_dis()(ei[1])
    dis2d = dis_pad.reshape(NPAD, 1)

    xp = jnp.pad(x, ((0, NPAD - N), (0, 0)))
    y = _tc1(dis2d, xp, W1)
    p = _make_agg(H)(y, src3, dst3, jnp.zeros((RPT, H), jnp.float32))

    W2p = jnp.pad(W2, ((0, 0), (0, CP - C)))
    b1r = b1.reshape(1, H)
    z = _tc2(dis2d, p, y, b1r, W2p)

    q = _make_agg(CP)(z, src3, dst3, jnp.zeros((RPT, CP), jnp.float32))
    b2r = jnp.concatenate([b2, jnp.full((CP - C,), -1e30, jnp.float32)]).reshape(1, CP)
    o = _tc3(dis2d, q, z, b2r)
    return o[:N, :C]


# speedup vs baseline: 32.3566x; 1.5077x over previous
import dataclasses
import functools

import jax
import jax.numpy as jnp
from jax import lax
from jax.experimental import pallas as pl
from jax.experimental.pallas import tpu as pltpu
from jax.experimental.pallas import tpu_sc as plsc

N = 10000
E = 320000
D = 128
H = 128
C = 40
CP = 48

NC = 2
NS = 16
NW = NC * NS
LANES = 16

NPAD = 10240

K = 80
EPT = E // NW
NCHUNK = EPT // K
RPT = NPAD // NS

EPT_DEG = E // NS
SL = NPAD // NS

_mesh = plsc.VectorSubcoreMesh(core_axis_name="c", subcore_axis_name="s")

_sc_params = pltpu.CompilerParams()
for _f, _v in (("needs_layout_passes", False), ("use_tc_tiling_on_sc", False)):
    if _f in pltpu.CompilerParams.__dataclass_fields__:
        _sc_params = dataclasses.replace(_sc_params, **{_f: _v})


def _deg_dis_kernel(dst_hbm, dis_hbm, dst_v, deg_v, acc_v, tmp_v, deg_sh):
    cid = lax.axis_index("c")
    sid = lax.axis_index("s")

    @pl.when(cid == 0)
    def _():
        @pl.loop(0, NPAD, step=LANES)
        def _(i):
            deg_v[pl.ds(i, LANES)] = jnp.zeros((LANES,), jnp.float32)

        pltpu.sync_copy(dst_hbm.at[pl.ds(sid * EPT_DEG, EPT_DEG)], dst_v)

        @pl.loop(0, EPT_DEG, step=LANES)
        def _(i):
            idx = dst_v[pl.ds(i, LANES)]
            plsc.addupdate_scatter(deg_v, [idx], jnp.ones((LANES,), jnp.float32))

        pltpu.sync_copy(deg_v, deg_sh.at[sid])

    plsc.subcore_barrier()

    @pl.when(cid == 0)
    def _():
        @pl.loop(0, SL, step=LANES)
        def _(i):
            acc_v[pl.ds(i, LANES)] = jnp.zeros((LANES,), jnp.float32)

        @pl.loop(0, NS)
        def _(k):
            pltpu.sync_copy(deg_sh.at[k, pl.ds(sid * SL, SL)], tmp_v)

            @pl.loop(0, SL, step=LANES)
            def _(i):
                acc_v[pl.ds(i, LANES)] = acc_v[pl.ds(i, LANES)] + tmp_v[pl.ds(i, LANES)]

        @pl.loop(0, SL, step=LANES)
        def _(i):
            d = acc_v[pl.ds(i, LANES)] + 1.0
            xh = d * 0.5
            ii = plsc.bitcast(d, jnp.int32)
            ii = jnp.int32(0x5F3759DF) - lax.shift_right_logical(ii, jnp.int32(1))
            yv = plsc.bitcast(ii, jnp.float32)
            yv = yv * (1.5 - xh * yv * yv)
            yv = yv * (1.5 - xh * yv * yv)
            yv = yv * (1.5 - xh * yv * yv)
            acc_v[pl.ds(i, LANES)] = yv

        pltpu.sync_copy(acc_v, dis_hbm.at[pl.ds(sid * SL, SL)])


def _make_deg_dis():
    return pl.kernel(
        _deg_dis_kernel,
        out_type=jax.ShapeDtypeStruct((NPAD,), jnp.float32),
        mesh=_mesh,
        scratch_types=[
            pltpu.VMEM((EPT_DEG,), jnp.int32),
            pltpu.VMEM((NPAD,), jnp.float32),
            pltpu.VMEM((SL,), jnp.float32),
            pltpu.VMEM((SL,), jnp.float32),
            pltpu.VMEM_SHARED((NS, NPAD), jnp.float32),
        ],
        compiler_params=_sc_params,
    )


RING = 2
MAIN = (NCHUNK // RING) * RING
NOUTER = MAIN // RING


def _agg_kernel(width, y_hbm, src_hbm, dst_hbm, zeros_hbm, out_hbm,
                src_v, dst_v, rows_v, acc_sh, gsem, ssem):
    cid = lax.axis_index("c")
    sid = lax.axis_index("s")
    wid = sid * NC + cid

    pltpu.sync_copy(src_hbm.at[wid], src_v)
    pltpu.sync_copy(dst_hbm.at[wid], dst_v)
    pltpu.sync_copy(zeros_hbm, acc_sh.at[pl.ds(sid * RPT, RPT)])
    plsc.subcore_barrier()

    for r in range(RING):
        pltpu.async_copy(y_hbm.at[src_v.at[r]], rows_v.at[r], gsem.at[r])

    @pl.loop(0, MAIN, step=RING)
    def _(j0):
        for r in range(RING):
            j = j0 + r
            pltpu.make_async_copy(
                y_hbm.at[src_v.at[j]], rows_v.at[r], gsem.at[r]).wait()
            pltpu.async_copy(
                rows_v.at[r], acc_sh.at[dst_v.at[j]], ssem.at[r], add=True).wait()

            @pl.when(j + RING < MAIN)
            def _():
                pltpu.async_copy(
                    y_hbm.at[src_v.at[j + RING]], rows_v.at[r], gsem.at[r])

    @pl.loop(MAIN, NCHUNK)
    def _(j):
        pltpu.sync_copy(y_hbm.at[src_v.at[j]], rows_v.at[0])
        pltpu.sync_copy(rows_v.at[0], acc_sh.at[dst_v.at[j]], add=True)

    plsc.subcore_barrier()
    pltpu.sync_copy(acc_sh.at[pl.ds(sid * RPT, RPT)],
                    out_hbm.at[cid, pl.ds(sid * RPT, RPT)])


def _make_agg(width):
    return pl.kernel(
        functools.partial(_agg_kernel, width),
        out_type=jax.ShapeDtypeStruct((NC, NPAD, width), jnp.float32),
        mesh=_mesh,
        scratch_types=[
            pltpu.VMEM((NCHUNK, K), jnp.int32),
            pltpu.VMEM((NCHUNK, K), jnp.int32),
            pltpu.VMEM((RING, K, width), jnp.float32),
            pltpu.VMEM_SHARED((NPAD, width), jnp.float32),
            pltpu.SemaphoreType.DMA((RING,)),
            pltpu.SemaphoreType.DMA((RING,)),
        ],
        compiler_params=_sc_params,
    )


RB = 2048
GRID = NPAD // RB


def _tc1_body(dis_ref, x_ref, w_ref, y_ref):
    xw = jnp.dot(x_ref[...], w_ref[...], preferred_element_type=jnp.float32)
    y_ref[...] = xw * dis_ref[...]


def _tc2_body(dis_ref, p_ref, y_ref, b_ref, w_ref, z_ref):
    pre = dis_ref[...] * (p_ref[0] + p_ref[1] + y_ref[...]) + b_ref[...]
    h = jnp.maximum(pre, 0.0)
    z_ref[...] = jnp.dot(h, w_ref[...], preferred_element_type=jnp.float32) * dis_ref[...]


def _tc3_body(dis_ref, q_ref, z_ref, b_ref, o_ref):
    logits = dis_ref[...] * (q_ref[0] + q_ref[1] + z_ref[...]) + b_ref[...]
    m = jnp.max(logits, axis=1, keepdims=True)
    e = logits - m
    o_ref[...] = e - jnp.log(jnp.sum(jnp.exp(e), axis=1, keepdims=True))


def _tc1(dis2d, x, W1):
    return pl.pallas_call(
        _tc1_body,
        grid=(GRID,),
        in_specs=[
            pl.BlockSpec((RB, 1), lambda i: (i, 0)),
            pl.BlockSpec((RB, D), lambda i: (i, 0)),
            pl.BlockSpec((D, H), lambda i: (0, 0)),
        ],
        out_specs=pl.BlockSpec((RB, H), lambda i: (i, 0)),
        out_shape=jax.ShapeDtypeStruct((NPAD, H), jnp.float32),
    )(dis2d, x, W1)


def _tc2(dis2d, p, y, b1r, W2p):
    return pl.pallas_call(
        _tc2_body,
        grid=(GRID,),
        in_specs=[
            pl.BlockSpec((RB, 1), lambda i: (i, 0)),
            pl.BlockSpec((NC, RB, H), lambda i: (0, i, 0)),
            pl.BlockSpec((RB, H), lambda i: (i, 0)),
            pl.BlockSpec((1, H), lambda i: (0, 0)),
            pl.BlockSpec((H, CP), lambda i: (0, 0)),
        ],
        out_specs=pl.BlockSpec((RB, CP), lambda i: (i, 0)),
        out_shape=jax.ShapeDtypeStruct((NPAD, CP), jnp.float32),
    )(dis2d, p, y, b1r, W2p)


def _tc3(dis2d, q, z, b2r):
    return pl.pallas_call(
        _tc3_body,
        grid=(GRID,),
        in_specs=[
            pl.BlockSpec((RB, 1), lambda i: (i, 0)),
            pl.BlockSpec((NC, RB, CP), lambda i: (0, i, 0)),
            pl.BlockSpec((RB, CP), lambda i: (i, 0)),
            pl.BlockSpec((1, CP), lambda i: (0, 0)),
        ],
        out_specs=pl.BlockSpec((RB, CP), lambda i: (i, 0)),
        out_shape=jax.ShapeDtypeStruct((NPAD, CP), jnp.float32),
    )(dis2d, q, z, b2r)


def kernel(x, edge_index, W1, b1, W2, b2):
    ei = edge_index.astype(jnp.int32)
    src3 = ei[0].reshape(NW, NCHUNK, K)
    dst3 = ei[1].reshape(NW, NCHUNK, K)

    dis_pad = _make_deg_dis()(ei[1])
    dis2d = dis_pad.reshape(NPAD, 1)

    xp = jnp.pad(x, ((0, NPAD - N), (0, 0)))
    y = _tc1(dis2d, xp, W1)
    p = _make_agg(H)(y, src3, dst3, jnp.zeros((RPT, H), jnp.float32))

    W2p = jnp.pad(W2, ((0, 0), (0, CP - C)))
    b1r = b1.reshape(1, H)
    z = _tc2(dis2d, p, y, b1r, W2p)

    q = _make_agg(CP)(z, src3, dst3, jnp.zeros((RPT, CP), jnp.float32))
    b2r = jnp.concatenate([b2, jnp.full((CP - C,), -1e30, jnp.float32)]).reshape(1, CP)
    o = _tc3(dis2d, q, z, b2r)
    return o[:N, :C]
